# baseline (speedup 1.0000x reference)
import jax
import jax.numpy as jnp
from jax.experimental import pallas as pl
from jax.experimental.pallas import tpu as pltpu
from jax.experimental.pallas import tpu_sc as plsc

_W = 32
_PAD = 128
_NTILES = 32


def kernel(indices, weight):
    B, L = indices.shape
    V, D = weight.shape
    N = B * L
    NP = N // 2

    pair_table = jnp.concatenate(
        [jnp.broadcast_to(weight[:, None, :], (V, V, D)),
         jnp.broadcast_to(weight[None, :, :], (V, V, D))],
        axis=-1).reshape(V * V, 2 * D)

    flat = indices.reshape(NP, 2)
    pair_idx = flat[:, 0] * V + flat[:, 1]
    idx_pad = jnp.pad(pair_idx.reshape(NP // _W, _W),
                      ((0, 0), (0, _PAD - _W)))

    n_win = NP // _W
    wpt = n_win // _NTILES

    mesh = plsc.VectorSubcoreMesh(core_axis_name="core",
                                  subcore_axis_name="subcore")

    @pl.kernel(out_type=jax.ShapeDtypeStruct((NP, 2 * D), weight.dtype),
               mesh=mesh)
    def sc_gather(w_hbm, i_hbm, o_hbm):
        def body(i_vmem, o_vmem):
            pltpu.sync_copy(w_hbm.at[i_vmem.at[0, pl.ds(0, _W)]], o_vmem)

        pltpu.emit_pipeline(
            body,
            grid=(_NTILES, wpt),
            in_specs=[pl.BlockSpec((1, _PAD),
                                   index_map=lambda c, i: (c * wpt + i, 0))],
            out_specs=[pl.BlockSpec((_W, 2 * D),
                                    index_map=lambda c, i: (c * wpt + i, 0))],
            core_axis_name=("core", "subcore"),
            dimension_semantics=(pltpu.PARALLEL, pltpu.ARBITRARY),
        )(i_hbm, o_hbm)

    out = sc_gather(pair_table, idx_pad)
    return out.reshape(B, L, D)
